# baseline (speedup 1.0000x reference)
import functools

import jax
import jax.numpy as jnp
from jax.experimental import pallas as pl
from jax.experimental.pallas import tpu as pltpu


def _router_body(x_ref, wr_ref, out_ref):
    out_ref[...] = jax.lax.dot_general(
        x_ref[...].astype(jnp.bfloat16), wr_ref[...].astype(jnp.bfloat16),
        (((1,), (1,)), ((), ())),
        preferred_element_type=jnp.float32)


def _moe_dense_body(x_ref, we_ref, be_ref, w_ref, out_ref, *, n_experts):
    xb = x_ref[...]
    acc = None
    for e in range(n_experts):
        pe = jax.lax.dot_general(
            xb, we_ref[e], (((1,), (1,)), ((), ())),
            preferred_element_type=jnp.float32)
        pe = (pe + be_ref[e]) * w_ref[0, :, e:e + 1]
        acc = pe if acc is None else acc + pe
    out_ref[...] = acc


def kernel(x, Wr, We, be):
    N, D = x.shape
    E = We.shape[0]
    TOPK = 2

    BN = 256
    T = 4
    DT = D // T
    NB = N // BN

    xb16 = x.astype(jnp.bfloat16)
    web16 = We.astype(jnp.bfloat16)

    logits = pl.pallas_call(
        _router_body,
        grid=(NB,),
        in_specs=[
            pl.BlockSpec((BN, D), lambda i: (i, 0)),
            pl.BlockSpec((E, D), lambda i: (0, 0)),
        ],
        out_specs=pl.BlockSpec((BN, E), lambda i: (i, 0)),
        out_shape=jax.ShapeDtypeStruct((N, E), jnp.float32),
    )(x, Wr)

    rw = jax.nn.softmax(logits, axis=1)
    topw, sel = jax.lax.top_k(rw, TOPK)
    topw = topw / jnp.sum(topw, axis=1, keepdims=True)

    w = jnp.zeros((N, E), jnp.float32)
    w = w.at[jnp.arange(N)[:, None], sel].add(topw)
    w = w.reshape(NB, BN, E)

    be3 = be.reshape(E, 1, D)

    out = pl.pallas_call(
        functools.partial(_moe_dense_body, n_experts=E),
        grid=(T, NB),
        in_specs=[
            pl.BlockSpec((BN, D), lambda t, i: (i, 0)),
            pl.BlockSpec((E, DT, D), lambda t, i: (0, t, 0)),
            pl.BlockSpec((E, 1, DT), lambda t, i: (0, 0, t)),
            pl.BlockSpec((1, BN, E), lambda t, i: (i, 0, 0)),
        ],
        out_specs=pl.BlockSpec((BN, DT), lambda t, i: (i, t)),
        out_shape=jax.ShapeDtypeStruct((N, D), jnp.float32),
    )(xb16, web16, be3, w)
    return out
